# baseline (speedup 1.0000x reference)
import functools

import jax
import jax.numpy as jnp
from jax import lax
from jax.experimental import pallas as pl
from jax.experimental.pallas import tpu as pltpu
from jax.experimental.pallas import tpu_sc as plsc

_NC, _NS = 2, 16
_NW = _NC * _NS
_BLK = 128
_DBLK = 120
_SB = 8
_T = 4000
_C = 3


def _sc_mesh():
    return plsc.VectorSubcoreMesh(core_axis_name="c", subcore_axis_name="s",
                                  num_cores=_NC, num_subcores=_NS)


def _make_routefetch(in_dim, npad, base, crows):
    nblk = crows // _DBLK
    nsteps = (nblk + _NW - 1) // _NW
    scratch = [
        pltpu.VMEM((2, _DBLK), jnp.int32),
        pltpu.VMEM((2, _DBLK, in_dim), jnp.float32),
        pltpu.SemaphoreType.DMA,
        pltpu.SemaphoreType.DMA,
        pltpu.SemaphoreType.DMA,
        pltpu.SemaphoreType.DMA,
        pltpu.SemaphoreType.DMA,
        pltpu.SemaphoreType.DMA,
    ]

    @functools.partial(
        pl.kernel,
        out_type=jax.ShapeDtypeStruct((crows, in_dim), jnp.float32),
        mesh=_sc_mesh(),
        scratch_types=scratch,
    )
    def fetch(x_hbm, gidx_hbm, xs_hbm, idx_v, rows_v,
              is0, is1, gs0, gs1, ws0, ws1):
        wid = lax.axis_index("s") * _NC + lax.axis_index("c")
        isem = (is0, is1)
        gsem = (gs0, gs1)
        wsem = (ws0, ws1)

        def idx_desc(j, p):
            off = (wid + _NW * j) * _DBLK
            return pltpu.make_async_copy(gidx_hbm.at[pl.ds(base + off, _DBLK)],
                                         idx_v.at[p], isem[p])

        def gath_desc(p):
            return pltpu.make_async_copy(x_hbm.at[idx_v.at[p]],
                                         rows_v.at[p], gsem[p])

        def write_desc(j, p):
            off = (wid + _NW * j) * _DBLK
            return pltpu.make_async_copy(rows_v.at[p],
                                         xs_hbm.at[pl.ds(off, _DBLK), :],
                                         wsem[p])

        @pl.when(wid < nblk)
        def _():
            idx_desc(0, 0).start()

        def half_step(j, p):
            b = wid + _NW * j

            @pl.when(jnp.logical_and(j >= 1, b - _NW < nblk))
            def _():
                write_desc(j - 1, 1 - p).wait()

            @pl.when(b + _NW < nblk)
            def _():
                idx_desc(j + 1, 1 - p).start()

            @pl.when(b < nblk)
            def _():
                idx_desc(j, p).wait()
                gath_desc(p).start()
                gath_desc(p).wait()
                write_desc(j, p).start()

        def step(jp, carry):
            half_step(2 * jp, 0)
            half_step(2 * jp + 1, 1)
            return carry

        lax.fori_loop(0, (nsteps + 2) // 2, step, 0)

    return fetch


def _make_writeback(nsb, npad):
    scratch = [
        pltpu.VMEM((_SB, _BLK), jnp.int32),
        pltpu.VMEM((_SB, _BLK), jnp.float32),
        pltpu.SemaphoreType.DMA,
    ]
    ksteps = (nsb + _NW - 1) // _NW

    @functools.partial(
        pl.kernel,
        out_type=jax.ShapeDtypeStruct((nsb, _SB, _BLK), jnp.float32),
        mesh=_sc_mesh(),
        scratch_types=scratch,
    )
    def writeback(ys_hbm, pos3_hbm, out_hbm, idx_v, y_v, sem):
        wid = lax.axis_index("s") * _NC + lax.axis_index("c")

        def step(k, carry):
            s = wid + _NW * k

            @pl.when(s < nsb)
            def _():
                pltpu.sync_copy(pos3_hbm.at[s], idx_v)
                for kk in range(_SB):
                    pltpu.make_async_copy(ys_hbm.at[idx_v.at[kk]],
                                          y_v.at[kk], sem).start()
                for kk in range(_SB):
                    pltpu.make_async_copy(ys_hbm.at[idx_v.at[kk]],
                                          y_v.at[kk], sem).wait()
                pltpu.sync_copy(y_v, out_hbm.at[s])

            return carry

        lax.fori_loop(0, ksteps, step, 0)

    return writeback


def _mlp_body(te_ref, xs_ref, w1_ref, b1_ref, w2_ref, b2_ref, o_ref):
    xb = xs_ref[...].astype(jnp.bfloat16)
    hT = lax.dot_general(w1_ref[0], xb, (((1,), (1,)), ((), ())),
                         preferred_element_type=jnp.float32)
    hT = jnp.maximum(hT + b1_ref[0, 0][:, None], 0.0).astype(jnp.bfloat16)
    oT = lax.dot_general(w2_ref[0, 0][None, :], hT, (((1,), (0,)), ((), ())),
                         preferred_element_type=jnp.float32)
    o_ref[0] = oT + b2_ref[0, 0, 0]


def kernel(x, atomic_nums, Wc1, bc1, Wc2, bc2, Wh1, bh1, Wh2, bh2,
           Wo1, bo1, Wo2, bo2):
    n, in_dim = x.shape
    hid = Wc1.shape[0]
    ntiles = (n + _T - 1) // _T + 2
    npad = ntiles * _T
    assert ntiles % _C == 0 and npad % (_C * _DBLK) == 0
    ctiles = ntiles // _C
    crows = npad // _C
    sbrows = _SB * _BLK
    nsb = (n + sbrows - 1) // sbrows
    ngpad = nsb * sbrows

    an = atomic_nums.astype(jnp.int32)
    is0 = an == 6
    is1 = an == 1
    c0 = jnp.sum(is0.astype(jnp.int32))
    c1 = jnp.sum(is1.astype(jnp.int32))
    s1 = ((c0 + _T - 1) // _T) * _T
    s2 = s1 + ((c1 + _T - 1) // _T) * _T
    cum0 = jnp.cumsum(is0.astype(jnp.int32))
    cum1 = jnp.cumsum(is1.astype(jnp.int32))
    iota1 = jnp.arange(1, n + 1, dtype=jnp.int32)
    pos = jnp.where(is0, cum0 - 1,
                    jnp.where(is1, s1 + cum1 - 1,
                              s2 + (iota1 - cum0 - cum1) - 1)).astype(jnp.int32)
    gidx = jnp.zeros((npad,), jnp.int32).at[pos].set(
        jnp.arange(n, dtype=jnp.int32), unique_indices=True,
        mode="promise_in_bounds")
    tstart = jnp.arange(ntiles, dtype=jnp.int32) * _T
    te = ((tstart >= s1).astype(jnp.int32) + (tstart >= s2).astype(jnp.int32))
    pos3 = jnp.concatenate(
        [pos, jnp.full((ngpad - n,), npad - 1, jnp.int32)]).reshape(nsb, _SB, _BLK)

    w1s = jnp.stack([Wc1, Wh1, Wo1]).astype(jnp.bfloat16)
    b1s = jnp.stack([bc1, bh1, bo1]).reshape(3, 1, hid)
    w2s = jnp.stack([Wc2[0], Wh2[0], Wo2[0]]).astype(jnp.bfloat16).reshape(3, 1, hid)
    b2s = jnp.stack([bc2, bh2, bo2]).reshape(3, 1, 1)

    grid_spec = pltpu.PrefetchScalarGridSpec(
        num_scalar_prefetch=1,
        grid=(ctiles,),
        in_specs=[
            pl.BlockSpec((_T, in_dim), lambda i, te_r: (i, 0)),
            pl.BlockSpec((1, hid, in_dim), lambda i, te_r: (te_r[i], 0, 0)),
            pl.BlockSpec((1, 1, hid), lambda i, te_r: (te_r[i], 0, 0)),
            pl.BlockSpec((1, 1, hid), lambda i, te_r: (te_r[i], 0, 0)),
            pl.BlockSpec((1, 1, 1), lambda i, te_r: (te_r[i], 0, 0)),
        ],
        out_specs=pl.BlockSpec((1, 1, _T), lambda i, te_r: (i, 0, 0)),
    )

    def tc_call(te_c, xs_c):
        return pl.pallas_call(
            _mlp_body,
            grid_spec=grid_spec,
            compiler_params=pltpu.CompilerParams(
                dimension_semantics=("arbitrary",)),
            out_shape=jax.ShapeDtypeStruct((ctiles, 1, _T), jnp.float32),
        )(te_c, xs_c, w1s, b1s, w2s, b2s)

    ys_chunks = []
    for c in range(_C):
        xs_c = _make_routefetch(in_dim, npad, c * crows, crows)(x, gidx)
        te_c = lax.dynamic_slice_in_dim(te, c * ctiles, ctiles)
        ys_chunks.append(tc_call(te_c, xs_c))
    ys = jnp.concatenate(ys_chunks).reshape(npad)

    out3 = _make_writeback(nsb, npad)(ys, pos3)
    return out3.reshape(ngpad)[:n].reshape(n, 1)


# speedup vs baseline: 1.1184x; 1.1184x over previous
import functools

import jax
import jax.numpy as jnp
from jax import lax
from jax.experimental import pallas as pl
from jax.experimental.pallas import tpu as pltpu
from jax.experimental.pallas import tpu_sc as plsc

_NC, _NS = 2, 16
_NW = _NC * _NS
_BLK = 128
_DBLK = 128
_SB = 8
_T = 4096
_C = 3


def _sc_mesh():
    return plsc.VectorSubcoreMesh(core_axis_name="c", subcore_axis_name="s",
                                  num_cores=_NC, num_subcores=_NS)


def _make_invperm(nsb, npad):
    scratch = [
        pltpu.VMEM((_SB, _BLK), jnp.int32),
        pltpu.VMEM((_SB, _BLK), jnp.int32),
        pltpu.SemaphoreType.DMA,
    ]
    ksteps = (nsb + _NW - 1) // _NW

    @functools.partial(
        pl.kernel,
        out_type=jax.ShapeDtypeStruct((npad,), jnp.int32),
        mesh=_sc_mesh(),
        scratch_types=scratch,
    )
    def invperm(pos3_hbm, iota3_hbm, gidx_hbm, idx_v, val_v, sem):
        wid = lax.axis_index("s") * _NC + lax.axis_index("c")

        def step(k, carry):
            s = wid + _NW * k

            @pl.when(s < nsb)
            def _():
                pltpu.sync_copy(pos3_hbm.at[s], idx_v)
                pltpu.sync_copy(iota3_hbm.at[s], val_v)
                for kk in range(_SB):
                    pltpu.make_async_copy(val_v.at[kk],
                                          gidx_hbm.at[idx_v.at[kk]], sem).start()
                for kk in range(_SB):
                    pltpu.make_async_copy(val_v.at[kk],
                                          gidx_hbm.at[idx_v.at[kk]], sem).wait()

            return carry

        lax.fori_loop(0, ksteps, step, 0)

    return invperm


def _make_routefetch(n, in_dim, npad, base, crows):
    nblk = crows // _DBLK
    nsteps = (nblk + _NW - 1) // _NW
    scratch = [
        pltpu.VMEM((2, _DBLK), jnp.int32),
        pltpu.VMEM((2, _DBLK, in_dim), jnp.float32),
        pltpu.SemaphoreType.DMA,
        pltpu.SemaphoreType.DMA,
        pltpu.SemaphoreType.DMA,
        pltpu.SemaphoreType.DMA,
        pltpu.SemaphoreType.DMA,
        pltpu.SemaphoreType.DMA,
    ]

    @functools.partial(
        pl.kernel,
        out_type=jax.ShapeDtypeStruct((crows, in_dim), jnp.float32),
        mesh=_sc_mesh(),
        scratch_types=scratch,
    )
    def fetch(x_hbm, gidx_hbm, xs_hbm, idx_v, rows_v,
              is0, is1, gs0, gs1, ws0, ws1):
        wid = lax.axis_index("s") * _NC + lax.axis_index("c")
        isem = (is0, is1)
        gsem = (gs0, gs1)
        wsem = (ws0, ws1)

        def idx_desc(j, p):
            off = (wid + _NW * j) * _DBLK
            return pltpu.make_async_copy(gidx_hbm.at[pl.ds(base + off, _DBLK)],
                                         idx_v.at[p], isem[p])

        def gath_desc(p):
            return pltpu.make_async_copy(x_hbm.at[idx_v.at[p]],
                                         rows_v.at[p], gsem[p])

        def write_desc(j, p):
            off = (wid + _NW * j) * _DBLK
            return pltpu.make_async_copy(rows_v.at[p],
                                         xs_hbm.at[pl.ds(off, _DBLK), :],
                                         wsem[p])

        @pl.when(wid < nblk)
        def _():
            idx_desc(0, 0).start()

        def half_step(j, p):
            b = wid + _NW * j

            @pl.when(jnp.logical_and(j >= 1, b - _NW < nblk))
            def _():
                write_desc(j - 1, 1 - p).wait()

            @pl.when(b + _NW < nblk)
            def _():
                idx_desc(j + 1, 1 - p).start()

            @pl.when(b < nblk)
            def _():
                idx_desc(j, p).wait()
                for r in range(_DBLK // 16):
                    sl = pl.ds(r * 16, 16)
                    v = idx_v[p, sl]
                    idx_v[p, sl] = jnp.minimum(jnp.maximum(v, 0), n - 1)
                gath_desc(p).start()
                gath_desc(p).wait()
                write_desc(j, p).start()

        def step(jp, carry):
            half_step(2 * jp, 0)
            half_step(2 * jp + 1, 1)
            return carry

        lax.fori_loop(0, (nsteps + 2) // 2, step, 0)

    return fetch


def _make_writeback(nsb, npad):
    scratch = [
        pltpu.VMEM((_SB, _BLK), jnp.int32),
        pltpu.VMEM((_SB, _BLK), jnp.float32),
        pltpu.SemaphoreType.DMA,
    ]
    ksteps = (nsb + _NW - 1) // _NW

    @functools.partial(
        pl.kernel,
        out_type=jax.ShapeDtypeStruct((nsb, _SB, _BLK), jnp.float32),
        mesh=_sc_mesh(),
        scratch_types=scratch,
    )
    def writeback(ys_hbm, pos3_hbm, out_hbm, idx_v, y_v, sem):
        wid = lax.axis_index("s") * _NC + lax.axis_index("c")

        def step(k, carry):
            s = wid + _NW * k

            @pl.when(s < nsb)
            def _():
                pltpu.sync_copy(pos3_hbm.at[s], idx_v)
                for kk in range(_SB):
                    pltpu.make_async_copy(ys_hbm.at[idx_v.at[kk]],
                                          y_v.at[kk], sem).start()
                for kk in range(_SB):
                    pltpu.make_async_copy(ys_hbm.at[idx_v.at[kk]],
                                          y_v.at[kk], sem).wait()
                pltpu.sync_copy(y_v, out_hbm.at[s])

            return carry

        lax.fori_loop(0, ksteps, step, 0)

    return writeback


def _mlp_body(te_ref, xs_ref, w1_ref, b1_ref, w2_ref, b2_ref, o_ref):
    xb = xs_ref[...].astype(jnp.bfloat16)
    hT = lax.dot_general(w1_ref[0], xb, (((1,), (1,)), ((), ())),
                         preferred_element_type=jnp.float32)
    hT = jnp.maximum(hT + b1_ref[0, 0][:, None], 0.0).astype(jnp.bfloat16)
    oT = lax.dot_general(w2_ref[0, 0][None, :], hT, (((1,), (0,)), ((), ())),
                         preferred_element_type=jnp.float32)
    o_ref[0] = oT + b2_ref[0, 0, 0]


def kernel(x, atomic_nums, Wc1, bc1, Wc2, bc2, Wh1, bh1, Wh2, bh2,
           Wo1, bo1, Wo2, bo2):
    n, in_dim = x.shape
    hid = Wc1.shape[0]
    ntiles = (n + _T - 1) // _T + 2
    npad = ntiles * _T
    assert ntiles % _C == 0 and npad % (_C * _DBLK) == 0
    ctiles = ntiles // _C
    crows = npad // _C
    sbrows = _SB * _BLK
    nsb = (n + sbrows - 1) // sbrows
    ngpad = nsb * sbrows

    an = atomic_nums.astype(jnp.int32)
    is0 = an == 6
    is1 = an == 1
    c0 = jnp.sum(is0.astype(jnp.int32))
    c1 = jnp.sum(is1.astype(jnp.int32))
    s1 = ((c0 + _T - 1) // _T) * _T
    s2 = s1 + ((c1 + _T - 1) // _T) * _T
    cum0 = jnp.cumsum(is0.astype(jnp.int32))
    cum1 = jnp.cumsum(is1.astype(jnp.int32))
    iota1 = jnp.arange(1, n + 1, dtype=jnp.int32)
    pos = jnp.where(is0, cum0 - 1,
                    jnp.where(is1, s1 + cum1 - 1,
                              s2 + (iota1 - cum0 - cum1) - 1)).astype(jnp.int32)
    tstart = jnp.arange(ntiles, dtype=jnp.int32) * _T
    te = ((tstart >= s1).astype(jnp.int32) + (tstart >= s2).astype(jnp.int32))
    pos3 = jnp.concatenate(
        [pos, jnp.full((ngpad - n,), npad - 1, jnp.int32)]).reshape(nsb, _SB, _BLK)
    iota3 = jnp.arange(ngpad, dtype=jnp.int32).reshape(nsb, _SB, _BLK)
    gidx = _make_invperm(nsb, npad)(pos3, iota3)

    w1s = jnp.stack([Wc1, Wh1, Wo1]).astype(jnp.bfloat16)
    b1s = jnp.stack([bc1, bh1, bo1]).reshape(3, 1, hid)
    w2s = jnp.stack([Wc2[0], Wh2[0], Wo2[0]]).astype(jnp.bfloat16).reshape(3, 1, hid)
    b2s = jnp.stack([bc2, bh2, bo2]).reshape(3, 1, 1)

    grid_spec = pltpu.PrefetchScalarGridSpec(
        num_scalar_prefetch=1,
        grid=(ctiles,),
        in_specs=[
            pl.BlockSpec((_T, in_dim), lambda i, te_r: (i, 0)),
            pl.BlockSpec((1, hid, in_dim), lambda i, te_r: (te_r[i], 0, 0)),
            pl.BlockSpec((1, 1, hid), lambda i, te_r: (te_r[i], 0, 0)),
            pl.BlockSpec((1, 1, hid), lambda i, te_r: (te_r[i], 0, 0)),
            pl.BlockSpec((1, 1, 1), lambda i, te_r: (te_r[i], 0, 0)),
        ],
        out_specs=pl.BlockSpec((1, 1, _T), lambda i, te_r: (i, 0, 0)),
    )

    def tc_call(te_c, xs_c):
        return pl.pallas_call(
            _mlp_body,
            grid_spec=grid_spec,
            compiler_params=pltpu.CompilerParams(
                dimension_semantics=("arbitrary",)),
            out_shape=jax.ShapeDtypeStruct((ctiles, 1, _T), jnp.float32),
        )(te_c, xs_c, w1s, b1s, w2s, b2s)

    ys_chunks = []
    for c in range(_C):
        xs_c = _make_routefetch(n, in_dim, npad, c * crows, crows)(x, gidx)
        te_c = lax.dynamic_slice_in_dim(te, c * ctiles, ctiles)
        ys_chunks.append(tc_call(te_c, xs_c))
    ys = jnp.concatenate(ys_chunks).reshape(npad)

    out3 = _make_writeback(nsb, npad)(ys, pos3)
    return out3.reshape(ngpad)[:n].reshape(n, 1)


# speedup vs baseline: 5.6533x; 5.0546x over previous
import functools

import jax
import jax.numpy as jnp
from jax import lax
from jax.experimental import pallas as pl
from jax.experimental.pallas import tpu as pltpu
from jax.experimental.pallas import tpu_sc as plsc

_NC, _NS = 2, 16
_NW = _NC * _NS
_BLK = 128
_SB = 8
_T = 4000


def _sc_mesh():
    return plsc.VectorSubcoreMesh(core_axis_name="c", subcore_axis_name="s",
                                  num_cores=_NC, num_subcores=_NS)


def _make_scatter(n, in_dim, npad, nfull, tail, nsteps):
    scratch = [
        pltpu.VMEM((2, _BLK), jnp.int32),
        pltpu.VMEM((2, _BLK, in_dim), jnp.float32),
        pltpu.VMEM((max(tail, 8),), jnp.int32),
        pltpu.VMEM((max(tail, 8), in_dim), jnp.float32),
        pltpu.SemaphoreType.DMA,
        pltpu.SemaphoreType.DMA,
        pltpu.SemaphoreType.DMA,
        pltpu.SemaphoreType.DMA,
        pltpu.SemaphoreType.DMA,
    ]

    @functools.partial(
        pl.kernel,
        out_type=jax.ShapeDtypeStruct((npad, in_dim), jnp.float32),
        mesh=_sc_mesh(),
        scratch_types=scratch,
    )
    def scatter(x_hbm, pos_hbm, xs_hbm, idx_v, rows_v, idxt_v, rowst_v,
                ls0, ls1, ss0, ss1, tsem):
        wid = lax.axis_index("s") * _NC + lax.axis_index("c")
        lsem = (ls0, ls1)
        ssem = (ss0, ss1)

        def load_descs(j, p):
            off = (wid + _NW * j) * _BLK
            di = pltpu.make_async_copy(pos_hbm.at[pl.ds(off, _BLK)],
                                       idx_v.at[p], lsem[p])
            dr = pltpu.make_async_copy(x_hbm.at[pl.ds(off, _BLK), :],
                                       rows_v.at[p], lsem[p])
            return di, dr

        def scat_desc(p):
            return pltpu.make_async_copy(rows_v.at[p], xs_hbm.at[idx_v.at[p]],
                                         ssem[p])

        @pl.when(wid < nfull)
        def _():
            di, dr = load_descs(0, 0)
            di.start()
            dr.start()

        def half_step(j, p):
            b = wid + _NW * j

            @pl.when(jnp.logical_and(j >= 1, b - _NW < nfull))
            def _():
                scat_desc(1 - p).wait()

            @pl.when(b + _NW < nfull)
            def _():
                di, dr = load_descs(j + 1, 1 - p)
                di.start()
                dr.start()

            @pl.when(b < nfull)
            def _():
                di, dr = load_descs(j, p)
                di.wait()
                dr.wait()
                scat_desc(p).start()

        def step(jp, carry):
            half_step(2 * jp, 0)
            half_step(2 * jp + 1, 1)
            return carry

        lax.fori_loop(0, (nsteps + 2) // 2, step, 0)

        if tail:
            @pl.when(wid == (nfull % _NW))
            def _():
                off = nfull * _BLK
                pltpu.sync_copy(pos_hbm.at[pl.ds(off, tail)],
                                idxt_v.at[pl.ds(0, tail)])
                pltpu.sync_copy(x_hbm.at[pl.ds(off, tail), :],
                                rowst_v.at[pl.ds(0, tail), :])
                pltpu.async_copy(rowst_v.at[pl.ds(0, tail), :],
                                 xs_hbm.at[idxt_v.at[pl.ds(0, tail)]],
                                 tsem).wait()

    return scatter


def _make_gather(nsb, npad):
    scratch = [
        pltpu.VMEM((_SB, _BLK), jnp.int32),
        pltpu.VMEM((_SB, _BLK), jnp.float32),
        pltpu.SemaphoreType.DMA,
    ]
    ksteps = (nsb + _NW - 1) // _NW

    @functools.partial(
        pl.kernel,
        out_type=jax.ShapeDtypeStruct((nsb, _SB, _BLK), jnp.float32),
        mesh=_sc_mesh(),
        scratch_types=scratch,
    )
    def gather(ys_hbm, pos3_hbm, out_hbm, idx_v, y_v, sem):
        wid = lax.axis_index("s") * _NC + lax.axis_index("c")

        def step(k, carry):
            s = wid + _NW * k

            @pl.when(s < nsb)
            def _():
                pltpu.sync_copy(pos3_hbm.at[s], idx_v)
                for kk in range(_SB):
                    pltpu.make_async_copy(ys_hbm.at[idx_v.at[kk]],
                                          y_v.at[kk], sem).start()
                for kk in range(_SB):
                    pltpu.make_async_copy(ys_hbm.at[idx_v.at[kk]],
                                          y_v.at[kk], sem).wait()
                pltpu.sync_copy(y_v, out_hbm.at[s])

            return carry

        lax.fori_loop(0, ksteps, step, 0)

    return gather


def _mlp_body(te_ref, xs_ref, w1_ref, b1_ref, w2_ref, b2_ref, o_ref):
    xb = xs_ref[...].astype(jnp.bfloat16)
    hT = lax.dot_general(w1_ref[0], xb, (((1,), (1,)), ((), ())),
                         preferred_element_type=jnp.float32)
    hT = jnp.maximum(hT + b1_ref[0, 0][:, None], 0.0).astype(jnp.bfloat16)
    oT = lax.dot_general(w2_ref[0, 0][None, :], hT, (((1,), (0,)), ((), ())),
                         preferred_element_type=jnp.float32)
    o_ref[0] = oT + b2_ref[0, 0, 0]


def kernel(x, atomic_nums, Wc1, bc1, Wc2, bc2, Wh1, bh1, Wh2, bh2,
           Wo1, bo1, Wo2, bo2):
    n, in_dim = x.shape
    hid = Wc1.shape[0]
    ntiles = (n + _T - 1) // _T + 2
    npad = ntiles * _T
    nfull = n // _BLK
    tail = n - nfull * _BLK
    nblocks = nfull + (1 if tail else 0)
    nsteps = (nblocks + _NW - 1) // _NW
    sbrows = _SB * _BLK
    nsb = (n + sbrows - 1) // sbrows
    ngpad = nsb * sbrows

    an = atomic_nums.astype(jnp.int32)
    is0 = an == 6
    is1 = an == 1
    c0 = jnp.sum(is0.astype(jnp.int32))
    c1 = jnp.sum(is1.astype(jnp.int32))
    s1 = ((c0 + _T - 1) // _T) * _T
    s2 = s1 + ((c1 + _T - 1) // _T) * _T
    cum0 = jnp.cumsum(is0.astype(jnp.int32))
    cum1 = jnp.cumsum(is1.astype(jnp.int32))
    iota1 = jnp.arange(1, n + 1, dtype=jnp.int32)
    pos = jnp.where(is0, cum0 - 1,
                    jnp.where(is1, s1 + cum1 - 1,
                              s2 + (iota1 - cum0 - cum1) - 1)).astype(jnp.int32)
    tstart = jnp.arange(ntiles, dtype=jnp.int32) * _T
    te = ((tstart >= s1).astype(jnp.int32) + (tstart >= s2).astype(jnp.int32))
    pos3 = jnp.concatenate(
        [pos, jnp.full((ngpad - n,), npad - 1, jnp.int32)]).reshape(nsb, _SB, _BLK)

    xs = _make_scatter(n, in_dim, npad, nfull, tail, nsteps)(x, pos)

    w1s = jnp.stack([Wc1, Wh1, Wo1]).astype(jnp.bfloat16)
    b1s = jnp.stack([bc1, bh1, bo1]).reshape(3, 1, hid)
    w2s = jnp.stack([Wc2[0], Wh2[0], Wo2[0]]).astype(jnp.bfloat16).reshape(3, 1, hid)
    b2s = jnp.stack([bc2, bh2, bo2]).reshape(3, 1, 1)

    grid_spec = pltpu.PrefetchScalarGridSpec(
        num_scalar_prefetch=1,
        grid=(ntiles,),
        in_specs=[
            pl.BlockSpec((_T, in_dim), lambda i, te_r: (i, 0)),
            pl.BlockSpec((1, hid, in_dim), lambda i, te_r: (te_r[i], 0, 0)),
            pl.BlockSpec((1, 1, hid), lambda i, te_r: (te_r[i], 0, 0)),
            pl.BlockSpec((1, 1, hid), lambda i, te_r: (te_r[i], 0, 0)),
            pl.BlockSpec((1, 1, 1), lambda i, te_r: (te_r[i], 0, 0)),
        ],
        out_specs=pl.BlockSpec((1, 1, _T), lambda i, te_r: (i, 0, 0)),
    )
    ys = pl.pallas_call(
        _mlp_body,
        grid_spec=grid_spec,
        compiler_params=pltpu.CompilerParams(
            dimension_semantics=("arbitrary",)),
        out_shape=jax.ShapeDtypeStruct((ntiles, 1, _T), jnp.float32),
    )(te, xs, w1s, b1s, w2s, b2s)
    ys = ys.reshape(npad)

    out3 = _make_gather(nsb, npad)(ys, pos3)
    return out3.reshape(ngpad)[:n].reshape(n, 1)
